# baseline (speedup 1.0000x reference)
import functools

import jax
import jax.numpy as jnp
from jax import lax
from jax.experimental import pallas as pl
from jax.experimental.pallas import tpu as pltpu
from jax.experimental.pallas import tpu_sc as plsc

V = 1000000
D = 20
B = 16384
BF = 2 * B
PD = 2 * D

_info = plsc.get_sparse_core_info()
_NC, _NS = _info.num_cores, _info.num_subcores
_NW = _NC * _NS
_B_PER_W = BF // _NW


def _make_gather():
    mesh = plsc.VectorSubcoreMesh(core_axis_name="c", subcore_axis_name="s")

    @functools.partial(
        pl.kernel,
        mesh=mesh,
        compiler_params=pltpu.CompilerParams(use_tc_tiling_on_sc=False),
        out_type=jax.ShapeDtypeStruct((BF, PD), jnp.float32),
        scratch_types=[
            pltpu.VMEM((_B_PER_W,), jnp.int32),
            pltpu.VMEM((_B_PER_W, PD), jnp.float32),
            pltpu.SemaphoreType.DMA,
        ],
    )
    def gather_k(table_hbm, idx_hbm, out_hbm, idx_v, rows_v, sem):
        wid = lax.axis_index("s") * _NC + lax.axis_index("c")
        base = wid * _B_PER_W
        pltpu.sync_copy(idx_hbm.at[pl.ds(base, _B_PER_W)], idx_v)
        pltpu.async_copy(table_hbm.at[idx_v], rows_v, sem).wait()
        pltpu.sync_copy(rows_v, out_hbm.at[pl.ds(base, _B_PER_W)])

    return gather_k


_gather = _make_gather()

_BLK = 2048


def _mlp_body(strips_ref, par_ref, wa_ref, wb_ref, b1_ref, w2_ref, b2_ref,
              out_ref):
    strip0 = strips_ref[:, :PD]
    strip1 = strips_ref[:, PD:]
    par = par_ref[...]
    p0 = par[:, 0:1]
    p1 = par[:, 1:2]
    z1 = jnp.broadcast_to(b1_ref[...], (_BLK, D))
    for s in range(2):
        pa = jnp.dot(strip0, wa_ref[s * PD:(s + 1) * PD, :],
                     preferred_element_type=jnp.float32)
        pb = jnp.dot(strip1, wb_ref[s * PD:(s + 1) * PD, :],
                     preferred_element_type=jnp.float32)
        z1 = z1 + jnp.where(p0 == s, pa, 0.0) + jnp.where(p1 == s, pb, 0.0)
    a1 = jax.nn.sigmoid(z1)
    z2 = jnp.sum(a1 * w2_ref[...], axis=1, keepdims=True) + b2_ref[...]
    out_ref[...] = jax.nn.sigmoid(z2)


def kernel(x, embedding, W1, b1, W2, b2):
    table = jnp.concatenate([embedding[0::2], embedding[1::2]], axis=1)
    v = x.astype(jnp.int32).reshape(-1)
    pidx = v // 2
    par = (v % 2).reshape(B, 2)

    rows = _gather(table, pidx)
    strips = windows = rows.reshape(B, 2 * PD)

    w1t = W1.T
    shifted_a = []
    shifted_b = []
    for s in range(2):
        sa = jnp.zeros((PD, D), jnp.float32)
        sa = sa.at[D * s:D * s + D, :].set(w1t[:D])
        sb = jnp.zeros((PD, D), jnp.float32)
        sb = sb.at[D * s:D * s + D, :].set(w1t[D:])
        shifted_a.append(sa)
        shifted_b.append(sb)
    wa = jnp.concatenate(shifted_a, axis=0)
    wb = jnp.concatenate(shifted_b, axis=0)

    grid = B // _BLK
    out = pl.pallas_call(
        _mlp_body,
        grid=(grid,),
        in_specs=[
            pl.BlockSpec((_BLK, 2 * PD), lambda i: (i, 0)),
            pl.BlockSpec((_BLK, 2), lambda i: (i, 0)),
            pl.BlockSpec((2 * PD, D), lambda i: (0, 0)),
            pl.BlockSpec((2 * PD, D), lambda i: (0, 0)),
            pl.BlockSpec((1, D), lambda i: (0, 0)),
            pl.BlockSpec((1, D), lambda i: (0, 0)),
            pl.BlockSpec((1, 1), lambda i: (0, 0)),
        ],
        out_specs=pl.BlockSpec((_BLK, 1), lambda i: (i, 0)),
        out_shape=jax.ShapeDtypeStruct((B, 1), jnp.float32),
    )(strips, par, wa, wb, b1.reshape(1, D), W2.reshape(1, D),
      b2.reshape(1, 1))
    return out


# speedup vs baseline: 1.7783x; 1.7783x over previous
import functools

import jax
import jax.numpy as jnp
from jax import lax
from jax.experimental import pallas as pl
from jax.experimental.pallas import tpu as pltpu
from jax.experimental.pallas import tpu_sc as plsc

V = 1000000
D = 20
B = 16384
BF = 2 * B
NSLOT = 5 * BF
NLIST = 6 * BF

_info = plsc.get_sparse_core_info()
_NC, _NS = _info.num_cores, _info.num_subcores
_NW = _NC * _NS
_ROWS_W = BF // _NW
_NDMA = 8
_ROWS_D = _ROWS_W // _NDMA
_SLOT_D = 5 * _ROWS_D
_LIST_D = 6 * _ROWS_D
_LIVE = 4
_ROUNDS = _NDMA // _LIVE


def _make_gather():
    mesh = plsc.VectorSubcoreMesh(core_axis_name="c", subcore_axis_name="s")

    scratch = [pltpu.VMEM((_LIST_D,), jnp.int32) for _ in range(_LIVE)]
    scratch += [pltpu.VMEM((_LIST_D, D), jnp.float32) for _ in range(_LIVE)]
    scratch += [pltpu.SemaphoreType.DMA]

    @functools.partial(
        pl.kernel,
        mesh=mesh,
        compiler_params=pltpu.CompilerParams(use_tc_tiling_on_sc=False),
        out_type=jax.ShapeDtypeStruct((NSLOT, D), jnp.float32),
        scratch_types=scratch,
    )
    def gather_k(table_hbm, idx_hbm, out_hbm, *refs):
        idx_refs = refs[:_LIVE]
        row_refs = refs[_LIVE:2 * _LIVE]
        sem = refs[2 * _LIVE]
        wid = lax.axis_index("s") * _NC + lax.axis_index("c")
        lbase = wid * _NDMA * _LIST_D
        obase = wid * _NDMA * _SLOT_D
        for r_ in range(_ROUNDS):
            for d_ in range(_LIVE):
                k_ = r_ * _LIVE + d_
                pltpu.sync_copy(
                    idx_hbm.at[pl.ds(lbase + k_ * _LIST_D, _LIST_D)],
                    idx_refs[d_])
            copies = [
                pltpu.async_copy(table_hbm.at[idx_refs[d_]], row_refs[d_], sem)
                for d_ in range(_LIVE)
            ]
            for c in copies:
                c.wait()
            for d_ in range(_LIVE):
                k_ = r_ * _LIVE + d_
                pltpu.sync_copy(
                    row_refs[d_].at[pl.ds(0, _SLOT_D)],
                    out_hbm.at[pl.ds(obase + k_ * _SLOT_D, _SLOT_D)])

    return gather_k


_gather = _make_gather()

_BLK = 2048


def _mlp_body(strips_ref, jj_ref, wa_ref, wb_ref, b1_ref, w2_ref, b2_ref,
              out_ref):
    strip0 = strips_ref[:, :5 * D]
    strip1 = strips_ref[:, 5 * D:]
    jj = jj_ref[...]
    j0 = jj[:, 0:1]
    j1 = jj[:, 1:2]
    z1 = jnp.broadcast_to(b1_ref[...], (_BLK, D))
    for s in range(5):
        pa = jnp.dot(strip0, wa_ref[s * 5 * D:(s + 1) * 5 * D, :],
                     preferred_element_type=jnp.float32)
        pb = jnp.dot(strip1, wb_ref[s * 5 * D:(s + 1) * 5 * D, :],
                     preferred_element_type=jnp.float32)
        z1 = z1 + jnp.where(j0 == s, pa, 0.0) + jnp.where(j1 == s, pb, 0.0)
    a1 = jax.nn.sigmoid(z1)
    z2 = jnp.sum(a1 * w2_ref[...], axis=1, keepdims=True) + b2_ref[...]
    out_ref[...] = jax.nn.sigmoid(z2)


def kernel(x, embedding, W1, b1, W2, b2):
    v = x.astype(jnp.int32).reshape(-1)
    j = v % 5
    L = (6 * v - j) // 5
    ar = jnp.arange(6, dtype=jnp.int32)[None, :]
    jc = j[:, None]
    entries = jnp.where(ar == jc, L[:, None], 0) + jnp.where(
        ar == jc + 1, L[:, None] + 1, 0)
    idx = entries.reshape(-1)
    jj = j.reshape(B, 2)

    slots = _gather(embedding, idx)
    strips = slots.reshape(B, 10 * D)

    w1t = W1.T
    shifted_a = []
    shifted_b = []
    for s in range(5):
        sa = jnp.zeros((5 * D, D), jnp.float32)
        sa = sa.at[D * s:D * s + D, :].set(w1t[:D])
        sb = jnp.zeros((5 * D, D), jnp.float32)
        sb = sb.at[D * s:D * s + D, :].set(w1t[D:])
        shifted_a.append(sa)
        shifted_b.append(sb)
    wa = jnp.concatenate(shifted_a, axis=0)
    wb = jnp.concatenate(shifted_b, axis=0)

    grid = B // _BLK
    out = pl.pallas_call(
        _mlp_body,
        grid=(grid,),
        in_specs=[
            pl.BlockSpec((_BLK, 10 * D), lambda i: (i, 0)),
            pl.BlockSpec((_BLK, 2), lambda i: (i, 0)),
            pl.BlockSpec((25 * D, D), lambda i: (0, 0)),
            pl.BlockSpec((25 * D, D), lambda i: (0, 0)),
            pl.BlockSpec((1, D), lambda i: (0, 0)),
            pl.BlockSpec((1, D), lambda i: (0, 0)),
            pl.BlockSpec((1, 1), lambda i: (0, 0)),
        ],
        out_specs=pl.BlockSpec((_BLK, 1), lambda i: (i, 0)),
        out_shape=jax.ShapeDtypeStruct((B, 1), jnp.float32),
    )(strips, jj, wa, wb, b1.reshape(1, D), W2.reshape(1, D),
      b2.reshape(1, 1))
    return out


# speedup vs baseline: 3.0989x; 1.7426x over previous
import functools

import jax
import jax.numpy as jnp
from jax import lax
from jax.experimental import pallas as pl
from jax.experimental.pallas import tpu as pltpu
from jax.experimental.pallas import tpu_sc as plsc

V = 1000000
D = 20
B = 16384
BF = 2 * B
NSLOT = 5 * BF
NLIST = 6 * BF

_info = plsc.get_sparse_core_info()
_NC, _NS = _info.num_cores, _info.num_subcores
_NW = _NC * _NS
_ROWS_W = BF // _NW
_NDMA = 8
_ROWS_D = _ROWS_W // _NDMA
_SLOT_D = 5 * _ROWS_D
_LIST_D = 6 * _ROWS_D
_LIVE = 4
_ROUNDS = _NDMA // _LIVE


def _make_gather():
    mesh = plsc.VectorSubcoreMesh(core_axis_name="c", subcore_axis_name="s")

    scratch = [pltpu.VMEM((_LIST_D,), jnp.int32) for _ in range(_LIVE)]
    scratch += [pltpu.VMEM((256,), jnp.int32)]
    scratch += [pltpu.VMEM((_LIST_D, D), jnp.float32) for _ in range(_LIVE)]
    scratch += [pltpu.SemaphoreType.DMA]

    @functools.partial(
        pl.kernel,
        mesh=mesh,
        compiler_params=pltpu.CompilerParams(use_tc_tiling_on_sc=False),
        out_type=jax.ShapeDtypeStruct((NSLOT, D), jnp.float32),
        scratch_types=scratch,
    )
    def gather_k(table_hbm, idx_hbm, out_hbm, *refs):
        idx_refs = refs[:_LIVE]
        guard = refs[_LIVE]
        row_refs = refs[_LIVE + 1:_LIVE + 1 + _LIVE]
        sem = refs[2 * _LIVE + 1]
        zeros16 = jnp.zeros((16,), jnp.int32)
        for g_ in range(16):
            guard[pl.ds(16 * g_, 16)] = zeros16
        wid = lax.axis_index("s") * _NC + lax.axis_index("c")
        lbase = wid * _NDMA * _LIST_D
        obase = wid * _NDMA * _SLOT_D
        for r_ in range(_ROUNDS):
            for d_ in range(_LIVE):
                k_ = r_ * _LIVE + d_
                pltpu.sync_copy(
                    idx_hbm.at[pl.ds(lbase + k_ * _LIST_D, _LIST_D)],
                    idx_refs[d_])
            copies = [
                pltpu.async_copy(table_hbm.at[idx_refs[d_]], row_refs[d_], sem)
                for d_ in range(_LIVE)
            ]
            for c in copies:
                c.wait()
            for d_ in range(_LIVE):
                k_ = r_ * _LIVE + d_
                pltpu.sync_copy(
                    row_refs[d_].at[pl.ds(0, _SLOT_D)],
                    out_hbm.at[pl.ds(obase + k_ * _SLOT_D, _SLOT_D)])

    return gather_k


_gather = _make_gather()

_BLK = 2048


def _mlp_body(strips_ref, jj_ref, wa_ref, wb_ref, b1_ref, w2_ref, b2_ref,
              out_ref):
    strip0 = strips_ref[:, :5 * D]
    strip1 = strips_ref[:, 5 * D:]
    jj = jj_ref[...]
    j0 = jj[:, 0:1]
    j1 = jj[:, 1:2]
    z1 = jnp.broadcast_to(b1_ref[...], (_BLK, D))
    for s in range(5):
        pa = jnp.dot(strip0, wa_ref[s * 5 * D:(s + 1) * 5 * D, :],
                     preferred_element_type=jnp.float32)
        pb = jnp.dot(strip1, wb_ref[s * 5 * D:(s + 1) * 5 * D, :],
                     preferred_element_type=jnp.float32)
        z1 = z1 + jnp.where(j0 == s, pa, 0.0) + jnp.where(j1 == s, pb, 0.0)
    a1 = jax.nn.sigmoid(z1)
    z2 = jnp.sum(a1 * w2_ref[...], axis=1, keepdims=True) + b2_ref[...]
    out_ref[...] = jax.nn.sigmoid(z2)


def kernel(x, embedding, W1, b1, W2, b2):
    v = x.astype(jnp.int32).reshape(-1)
    j = v % 5
    L = (6 * v - j) // 5
    ar = jnp.arange(6, dtype=jnp.int32)[None, :]
    jc = j[:, None]
    entries = jnp.where(ar == jc + 1, L[:, None] + 1, L[:, None])
    idx = entries.reshape(-1)
    jj = j.reshape(B, 2)

    slots = _gather(embedding, idx)
    strips = slots.reshape(B, 10 * D)

    w1t = W1.T
    shifted_a = []
    shifted_b = []
    for s in range(5):
        sa = jnp.zeros((5 * D, D), jnp.float32)
        sa = sa.at[D * s:D * s + D, :].set(w1t[:D])
        sb = jnp.zeros((5 * D, D), jnp.float32)
        sb = sb.at[D * s:D * s + D, :].set(w1t[D:])
        shifted_a.append(sa)
        shifted_b.append(sb)
    wa = jnp.concatenate(shifted_a, axis=0)
    wb = jnp.concatenate(shifted_b, axis=0)

    grid = B // _BLK
    out = pl.pallas_call(
        _mlp_body,
        grid=(grid,),
        in_specs=[
            pl.BlockSpec((_BLK, 10 * D), lambda i: (i, 0)),
            pl.BlockSpec((_BLK, 2), lambda i: (i, 0)),
            pl.BlockSpec((25 * D, D), lambda i: (0, 0)),
            pl.BlockSpec((25 * D, D), lambda i: (0, 0)),
            pl.BlockSpec((1, D), lambda i: (0, 0)),
            pl.BlockSpec((1, D), lambda i: (0, 0)),
            pl.BlockSpec((1, 1), lambda i: (0, 0)),
        ],
        out_specs=pl.BlockSpec((_BLK, 1), lambda i: (i, 0)),
        out_shape=jax.ShapeDtypeStruct((B, 1), jnp.float32),
    )(strips, jj, wa, wb, b1.reshape(1, D), W2.reshape(1, D),
      b2.reshape(1, 1))
    return out
